# baseline (speedup 1.0000x reference)
import functools

import jax
import jax.numpy as jnp
from jax import lax
from jax.experimental import pallas as pl
from jax.experimental.pallas import tpu as pltpu
from jax.experimental.pallas import tpu_sc as plsc

_S = 10000
_D = 128


def _prepool_body(x_ref, wt_ref, b_ref, h_ref):
    h_ref[...] = (
        jnp.dot(x_ref[...], wt_ref[...], preferred_element_type=jnp.float32)
        + b_ref[...]
    )


def _prepool(x, wt, b):
    n, d = x.shape
    blk = 2560
    return pl.pallas_call(
        _prepool_body,
        grid=(n // blk,),
        in_specs=[
            pl.BlockSpec((blk, d), lambda i: (i, 0)),
            pl.BlockSpec((d, d), lambda i: (0, 0)),
            pl.BlockSpec((1, d), lambda i: (0, 0)),
        ],
        out_specs=pl.BlockSpec((blk, d), lambda i: (i, 0)),
        out_shape=jax.ShapeDtypeStruct((n, d), jnp.float32),
    )(x, wt, b.reshape(1, d))


def _proj_body(st_ref, wt_ref, b_ref, o_ref):
    st = st_ref[...]
    mean = st[:, 0:128] * (1.0 / jnp.maximum(st[:, 256:384], 1.0))
    xpool = jnp.concatenate([mean, st[:, 128:256]], axis=1)
    o_ref[...] = (
        jnp.dot(xpool, wt_ref[...], preferred_element_type=jnp.float32)
        + b_ref[...]
    )


def _proj(stats, wt, b):
    s = stats.shape[0]
    blk = 2000
    return pl.pallas_call(
        _proj_body,
        grid=(s // blk,),
        in_specs=[
            pl.BlockSpec((blk, 3 * _D), lambda i: (i, 0)),
            pl.BlockSpec((2 * _D, _D), lambda i: (0, 0)),
            pl.BlockSpec((1, _D), lambda i: (0, 0)),
        ],
        out_specs=pl.BlockSpec((blk, _D), lambda i: (i, 0)),
        out_shape=jax.ShapeDtypeStruct((s, _D), jnp.float32),
    )(stats, wt, b.reshape(1, _D))


def _segstats(h, ids):
    n, d = h.shape
    nv = d // 16
    st = 3 * d
    info = plsc.get_sparse_core_info()
    nc, ns = info.num_cores, info.num_subcores
    nw = nc * ns
    cw = n // nw
    r = 128
    mesh = plsc.VectorSubcoreMesh(core_axis_name="c", subcore_axis_name="s")

    @functools.partial(
        pl.kernel,
        out_type=jax.ShapeDtypeStruct((_S, st), jnp.float32),
        mesh=mesh,
        compiler_params=pltpu.CompilerParams(needs_layout_passes=False),
        scratch_types=[
            pltpu.VMEM((r, d), jnp.float32),
            pltpu.VMEM((r, d), jnp.float32),
            pltpu.VMEM((r + 16,), jnp.int32),
            pltpu.VMEM((r + 16,), jnp.int32),
            pltpu.VMEM((32,), jnp.int32),
            pltpu.VMEM((2, st), jnp.float32),
            pltpu.VMEM((48,), jnp.int32),
            pltpu.SemaphoreType.DMA,
            pltpu.SemaphoreType.DMA,
            pltpu.SemaphoreType.DMA,
        ],
    )
    def body(h_hbm, ids_hbm, stats_hbm, rb0, rb1, ib0, ib1, cmpb, stage,
             probe, csem0, csem1, fsem):
        w = lax.axis_index("s") * nc + lax.axis_index("c")
        iota16 = lax.iota(jnp.int32, 16)
        rbufs = (rb0, rb1)
        ibufs = (ib0, ib1)
        csems = (csem0, csem1)

        def extract(v, off):
            return jnp.max(jnp.where(iota16 == off, v, jnp.int32(-1)))

        def find_boundary(t):
            def b_cond(stt):
                base, res = stt
                return (res < 0) & (base < n)

            def b_body(stt):
                base, _ = stt
                a = pl.multiple_of(
                    jnp.maximum(jnp.minimum((base - 1) // 8 * 8, n - 48), 0), 8
                )
                pltpu.sync_copy(ids_hbm.at[pl.ds(a, 48)], probe)
                j = jnp.minimum(base - a, 32)
                v = probe[pl.ds(j, 16)]
                vp = probe[pl.ds(j - 1, 16)]
                idxv = a + j + iota16
                cand = jnp.where(
                    (v != vp) & (idxv >= base) & (idxv < n), idxv, n
                )
                m = jnp.min(cand)
                res2 = jnp.where(
                    m < n, m, jnp.where(base + 16 >= n, n, -1)
                ).astype(jnp.int32)
                return (jnp.int32(base + 16), res2)

            _, res = lax.while_loop(
                b_cond,
                b_body,
                (jnp.maximum(jnp.int32(t), 1), jnp.int32(-1)),
            )
            return jnp.where(
                t <= 0, 0, jnp.where(t >= n, n, res)
            ).astype(jnp.int32)

        def load_id(i):
            sb = pl.multiple_of(jnp.minimum(i // 8 * 8, n - 48), 8)
            pltpu.sync_copy(ids_hbm.at[pl.ds(sb, 48)], probe)
            off = i - sb
            ob = jnp.minimum(off, 32)
            return extract(probe[pl.ds(ob, 16)], off - ob)

        start = find_boundary(w * cw)
        end = find_boundary((w + 1) * cw)
        astart = pl.multiple_of(start // 8 * 8, 8)
        nchunks = (end - astart + r - 1) // r

        def chunk_lb(k):
            cb = astart + k * r
            return cb, pl.multiple_of(jnp.minimum(cb, n - r), 8)

        def issue_chunk(k, par):
            _, lb = chunk_lb(k)
            pltpu.async_copy(h_hbm.at[pl.ds(lb, r)], rbufs[par], csems[par])
            pltpu.async_copy(
                ids_hbm.at[pl.ds(lb, r)], ibufs[par].at[pl.ds(0, r)],
                csems[par],
            )

        def wait_chunk(k, par):
            _, lb = chunk_lb(k)
            pltpu.make_async_copy(
                h_hbm.at[pl.ds(lb, r)], rbufs[par], csems[par]
            ).wait()
            pltpu.make_async_copy(
                ids_hbm.at[pl.ds(lb, r)], ibufs[par].at[pl.ds(0, r)],
                csems[par],
            ).wait()

        def flush_wait():
            pltpu.make_async_copy(stats_hbm.at[0], stage.at[1], fsem).wait()

        def flush(seg, cnt, sums, maxs):
            flush_wait()
            cv = jnp.zeros((16,), jnp.float32) + cnt.astype(jnp.float32)
            for i in range(nv):
                stage[0, pl.ds(i * 16, 16)] = sums[i]
                stage[0, pl.ds(d + i * 16, 16)] = maxs[i]
                stage[0, pl.ds(2 * d + i * 16, 16)] = cv
            pltpu.async_copy(stage.at[0], stats_hbm.at[seg], fsem)

        def accum(rowb, j0, j1, sums, maxs):
            def rbody(j, vs):
                ss, mm = vs
                ss2 = []
                mm2 = []
                for i in range(nv):
                    v = rowb[j, pl.ds(i * 16, 16)]
                    ss2.append(ss[i] + v)
                    mm2.append(jnp.maximum(mm[i], v))
                return (tuple(ss2), tuple(mm2))

            return lax.fori_loop(j0, j1, rbody, (sums, maxs))

        zero_vs = tuple(jnp.zeros((16,), jnp.float32) for _ in range(nv))
        neg_vs = tuple(
            jnp.full((16,), -3.4e38, jnp.float32) for _ in range(nv)
        )

        def process_chunk(k, par, carry):
            cur, cnt, sums, maxs = carry
            wait_chunk(k, par)

            @pl.when(k + 1 < nchunks)
            def _():
                issue_chunk(k + 1, par ^ 1)

            cb, lb = chunk_lb(k)
            rowb = rbufs[par]
            idbuf = ibufs[par]
            jlo = jnp.maximum(start, cb) - lb
            jhi = jnp.minimum(end, cb + r) - lb

            jb = jnp.minimum(jlo, r - 16)
            idlo = extract(idbuf[pl.ds(jb, 16)], jlo - jb)

            def cross_flush(c):
                cur2, cnt2, ss, mm = c
                flush(cur2, cnt2, ss, mm)
                return (idlo, jnp.int32(0), zero_vs, neg_vs)

            cur, cnt, sums, maxs = lax.cond(
                idlo != cur, cross_flush, lambda c: c,
                (cur, cnt, sums, maxs),
            )

            j = jlo
            for gg in range(r // 16):
                p0 = gg * 16
                v = idbuf[pl.ds(p0 + 1, 16)]
                vp = idbuf[pl.ds(p0, 16)]
                idxv = iota16 + (p0 + 1)
                mask = (v != vp) & (idxv > jlo) & (idxv < jhi)
                plsc.store_compressed(cmpb.at[pl.ds(0, 16)], idxv, mask=mask)
                pc = jnp.max(
                    plsc.all_reduce_population_count(mask)
                ).astype(jnp.int32)

                full = (pc == 0) & (p0 >= jlo) & (p0 + 16 <= jhi)

                def fast(c):
                    cur2, cnt2, _, ss, mm = c

                    def quad(qq, vs):
                        ss2, mm2 = vs
                        for rr in range(4):
                            row = qq * 4 + (p0 + rr)
                            ss3 = []
                            mm3 = []
                            for i in range(nv):
                                vv = rowb[row, pl.ds(i * 16, 16)]
                                ss3.append(ss2[i] + vv)
                                mm3.append(jnp.maximum(mm2[i], vv))
                            ss2 = tuple(ss3)
                            mm2 = tuple(mm3)
                        return (ss2, mm2)

                    ss, mm = lax.fori_loop(0, 4, quad, (ss, mm))
                    return (cur2, cnt2 + 16, jnp.int32(p0 + 16), ss, mm)

                def slow(c):
                    cur2, cnt2, j2, ss, mm = c

                    def seg_body(bi, c2):
                        cur3, cnt3, j3, ss2, mm2 = c2
                        b = extract(cmpb[pl.ds(jnp.minimum(bi, 15), 16)], 0)
                        ss2, mm2 = accum(rowb, j3, b, ss2, mm2)
                        cnt3 = cnt3 + (b - j3)
                        flush(cur3, cnt3, ss2, mm2)
                        sid = extract(idbuf[pl.ds(b, 16)], 0)
                        return (sid, jnp.int32(0), b, zero_vs, neg_vs)

                    cur2, cnt2, j2, ss, mm = lax.fori_loop(
                        0, pc, seg_body, (cur2, cnt2, j2, ss, mm)
                    )
                    g1 = jnp.minimum(jhi, p0 + 16)
                    ss, mm = accum(rowb, j2, g1, ss, mm)
                    cnt2 = cnt2 + jnp.maximum(g1 - j2, 0)
                    return (cur2, cnt2, jnp.maximum(j2, g1), ss, mm)

                cur, cnt, j, sums, maxs = lax.cond(
                    full, fast, slow, (cur, cnt, j, sums, maxs)
                )

            return (cur, cnt, sums, maxs)

        @pl.when(start < end)
        def _():
            cur0 = load_id(start)
            issue_chunk(0, 0)
            pltpu.async_copy(stats_hbm.at[0], stage.at[1], fsem)

            def pair(kk, carry):
                for par in (0, 1):
                    k = kk * 2 + par
                    carry = lax.cond(
                        k < nchunks,
                        functools.partial(process_chunk, k, par),
                        lambda c: c,
                        carry,
                    )
                return carry

            init = (cur0, jnp.int32(0), zero_vs, neg_vs)
            cur, cnt, sums, maxs = lax.fori_loop(
                0, (nchunks + 1) // 2, pair, init
            )
            flush(cur, cnt, sums, maxs)
            flush_wait()

    return body(h, ids)


def _gather(table, ids):
    n = ids.shape[0]
    s, d = table.shape
    nv = d // 16
    info = plsc.get_sparse_core_info()
    nc, ns = info.num_cores, info.num_subcores
    nw = nc * ns
    cw = n // nw
    t = 128
    wsz = 64
    nring = 4
    nch = (cw + t - 1) // t
    mesh = plsc.VectorSubcoreMesh(core_axis_name="c", subcore_axis_name="s")

    @functools.partial(
        pl.kernel,
        out_type=jax.ShapeDtypeStruct((n, d), jnp.float32),
        mesh=mesh,
        compiler_params=pltpu.CompilerParams(needs_layout_passes=False),
        scratch_types=[pltpu.VMEM((cw + 16,), jnp.int32)]
        + [pltpu.VMEM((wsz, d), jnp.float32) for _ in range(nring)]
        + [pltpu.VMEM((t, d), jnp.float32) for _ in range(nring)]
        + [pltpu.VMEM((32,), jnp.int32)]
        + [pltpu.SemaphoreType.DMA for _ in range(2 * nring + 1)],
    )
    def body(tab_hbm, ids_hbm, out_hbm, ixall, *bufs):
        winb = bufs[:nring]
        stgb = bufs[nring:2 * nring]
        cmpb = bufs[2 * nring]
        wsem = bufs[2 * nring + 1:3 * nring + 1]
        ssem = bufs[3 * nring + 1:4 * nring + 1]
        fbsem = bufs[4 * nring + 1]
        w = lax.axis_index("s") * nc + lax.axis_index("c")
        base0 = w * cw
        iota16 = lax.iota(jnp.int32, 16)

        def extract(v, off):
            return jnp.max(jnp.where(iota16 == off, v, jnp.int32(-1)))

        def lbase(k):
            return pl.multiple_of(jnp.minimum(k * t, cw - t), 8)

        def meta(k):
            j0 = lbase(k)
            idf = extract(ixall[pl.ds(j0, 16)], 0)
            idl = extract(ixall[pl.ds(j0 + t - 16, 16)], 15)
            wb = pl.multiple_of(jnp.minimum(idf, s - wsz) // 8 * 8, 8)
            ok = (idl - wb) < wsz
            return j0, wb, ok

        def issue_window(k, p):
            _, wb, ok = meta(k)

            @pl.when(ok)
            def _():
                pltpu.async_copy(
                    tab_hbm.at[pl.ds(wb, wsz)], winb[p], wsem[p]
                )

        def run_copy(stage, win, a, b, src):
            regs = [win[src, pl.ds(i * 16, 16)] for i in range(nv)]

            def rbody(jr, _):
                for i in range(nv):
                    stage[jr, pl.ds(i * 16, 16)] = regs[i]
                return 0

            lax.fori_loop(a, b, rbody, 0)

        def step(k, p):
            @pl.when(k >= nring)
            def _():
                pltpu.make_async_copy(
                    stgb[p], out_hbm.at[pl.ds(base0 + lbase(k - nring), t)],
                    ssem[p],
                ).wait()

            j0, wb, ok = meta(k)

            def expand(_):
                pltpu.make_async_copy(
                    tab_hbm.at[pl.ds(wb, wsz)], winb[p], wsem[p]
                ).wait()
                cursrc = extract(ixall[pl.ds(j0, 16)], 0) - wb
                j = jnp.int32(0)
                carry = (j, cursrc)
                for q8 in range(t // 16):
                    p1 = j0 + q8 * 16 + 1
                    v = ixall[pl.ds(p1, 16)]
                    vp = ixall[pl.ds(p1 - 1, 16)]
                    idxv = iota16 + (q8 * 16 + 1)
                    mask = (v != vp) & (idxv < t)
                    plsc.store_compressed(
                        cmpb.at[pl.ds(0, 16)], idxv, mask=mask
                    )
                    pc = jnp.max(
                        plsc.all_reduce_population_count(mask)
                    ).astype(jnp.int32)

                    def rloop(bi, c):
                        j2, src2 = c
                        b = extract(
                            cmpb[pl.ds(jnp.minimum(bi, 15), 16)], 0
                        )
                        run_copy(stgb[p], winb[p], j2, b, src2)
                        nsrc = extract(ixall[pl.ds(j0 + b, 16)], 0) - wb
                        return (b, nsrc)

                    carry = lax.fori_loop(0, pc, rloop, carry)
                j, cursrc = carry
                run_copy(stgb[p], winb[p], j, t, cursrc)
                return 0

            def fallback(_):
                pltpu.async_copy(
                    tab_hbm.at[ixall.at[pl.ds(j0, t)]], stgb[p], fbsem
                ).wait()
                return 0

            lax.cond(ok, expand, fallback, 0)
            pltpu.async_copy(
                stgb[p], out_hbm.at[pl.ds(base0 + j0, t)], ssem[p]
            )

            @pl.when(k + 3 < nch)
            def _():
                issue_window(k + 3, (p + 3) % nring)

        pltpu.sync_copy(ids_hbm.at[pl.ds(base0, cw)], ixall.at[pl.ds(0, cw)])

        for kp in range(min(3, nch)):
            issue_window(kp, kp)

        def ring(kk, _):
            for par in range(nring):
                k = kk * nring + par

                @pl.when(k < nch)
                def _():
                    step(k, par)

            return 0

        lax.fori_loop(0, (nch + nring - 1) // nring, ring, 0)
        for tail in range(max(nch - nring, 0), nch):
            pltpu.make_async_copy(
                stgb[tail % nring],
                out_hbm.at[pl.ds(base0 + lbase(tail), t)],
                ssem[tail % nring],
            ).wait()

    return body(table, ids)


def kernel(x, batch_index, Wpre, bpre, Wproj, bproj):
    ids = batch_index.astype(jnp.int32)
    h = _prepool(x, Wpre.T, bpre)
    stats = _segstats(h, ids)
    pooled = _proj(stats, Wproj.T, bproj)
    return _gather(pooled, ids)


# speedup vs baseline: 1.2225x; 1.2225x over previous
import functools

import jax
import jax.numpy as jnp
from jax import lax
from jax.experimental import pallas as pl
from jax.experimental.pallas import tpu as pltpu
from jax.experimental.pallas import tpu_sc as plsc

_S = 10000
_D = 128


def _prepool_body(x_ref, wt_ref, b_ref, h_ref):
    h_ref[...] = (
        jnp.dot(x_ref[...], wt_ref[...], preferred_element_type=jnp.float32)
        + b_ref[...]
    )


def _prepool(x, wt, b):
    n, d = x.shape
    blk = 6400
    return pl.pallas_call(
        _prepool_body,
        grid=(n // blk,),
        in_specs=[
            pl.BlockSpec((blk, d), lambda i: (i, 0)),
            pl.BlockSpec((d, d), lambda i: (0, 0)),
            pl.BlockSpec((1, d), lambda i: (0, 0)),
        ],
        out_specs=pl.BlockSpec((blk, d), lambda i: (i, 0)),
        out_shape=jax.ShapeDtypeStruct((n, d), jnp.float32),
    )(x, wt, b.reshape(1, d))


def _proj_body(st_ref, wt_ref, b_ref, o_ref):
    st = st_ref[...]
    mean = st[:, 0:128] * (1.0 / jnp.maximum(st[:, 256:384], 1.0))
    xpool = jnp.concatenate([mean, st[:, 128:256]], axis=1)
    o_ref[...] = (
        jnp.dot(xpool, wt_ref[...], preferred_element_type=jnp.float32)
        + b_ref[...]
    )


def _proj(stats, wt, b):
    s = stats.shape[0]
    blk = 2000
    return pl.pallas_call(
        _proj_body,
        grid=(s // blk,),
        in_specs=[
            pl.BlockSpec((blk, 3 * _D), lambda i: (i, 0)),
            pl.BlockSpec((2 * _D, _D), lambda i: (0, 0)),
            pl.BlockSpec((1, _D), lambda i: (0, 0)),
        ],
        out_specs=pl.BlockSpec((blk, _D), lambda i: (i, 0)),
        out_shape=jax.ShapeDtypeStruct((s, _D), jnp.float32),
    )(stats, wt, b.reshape(1, _D))


def _segstats(h, ids):
    n, d = h.shape
    nv = d // 16
    st = 3 * d
    info = plsc.get_sparse_core_info()
    nc, ns = info.num_cores, info.num_subcores
    nw = nc * ns
    cw = n // nw
    r = 256
    mesh = plsc.VectorSubcoreMesh(core_axis_name="c", subcore_axis_name="s")

    @functools.partial(
        pl.kernel,
        out_type=jax.ShapeDtypeStruct((_S, st), jnp.float32),
        mesh=mesh,
        compiler_params=pltpu.CompilerParams(needs_layout_passes=False),
        scratch_types=[
            pltpu.VMEM((r, d), jnp.float32),
            pltpu.VMEM((r, d), jnp.float32),
            pltpu.VMEM((r + 16,), jnp.int32),
            pltpu.VMEM((r + 16,), jnp.int32),
            pltpu.VMEM((32,), jnp.int32),
            pltpu.VMEM((2, st), jnp.float32),
            pltpu.VMEM((48,), jnp.int32),
            pltpu.SemaphoreType.DMA,
            pltpu.SemaphoreType.DMA,
            pltpu.SemaphoreType.DMA,
        ],
    )
    def body(h_hbm, ids_hbm, stats_hbm, rb0, rb1, ib0, ib1, cmpb, stage,
             probe, csem0, csem1, fsem):
        w = lax.axis_index("s") * nc + lax.axis_index("c")
        iota16 = lax.iota(jnp.int32, 16)
        rbufs = (rb0, rb1)
        ibufs = (ib0, ib1)
        csems = (csem0, csem1)

        def extract(v, off):
            return jnp.max(jnp.where(iota16 == off, v, jnp.int32(-1)))

        def find_boundary(t):
            def b_cond(stt):
                base, res = stt
                return (res < 0) & (base < n)

            def b_body(stt):
                base, _ = stt
                a = pl.multiple_of(
                    jnp.maximum(jnp.minimum((base - 1) // 8 * 8, n - 48), 0), 8
                )
                pltpu.sync_copy(ids_hbm.at[pl.ds(a, 48)], probe)
                j = jnp.minimum(base - a, 32)
                v = probe[pl.ds(j, 16)]
                vp = probe[pl.ds(j - 1, 16)]
                idxv = a + j + iota16
                cand = jnp.where(
                    (v != vp) & (idxv >= base) & (idxv < n), idxv, n
                )
                m = jnp.min(cand)
                res2 = jnp.where(
                    m < n, m, jnp.where(base + 16 >= n, n, -1)
                ).astype(jnp.int32)
                return (jnp.int32(base + 16), res2)

            _, res = lax.while_loop(
                b_cond,
                b_body,
                (jnp.maximum(jnp.int32(t), 1), jnp.int32(-1)),
            )
            return jnp.where(
                t <= 0, 0, jnp.where(t >= n, n, res)
            ).astype(jnp.int32)

        def load_id(i):
            sb = pl.multiple_of(jnp.minimum(i // 8 * 8, n - 48), 8)
            pltpu.sync_copy(ids_hbm.at[pl.ds(sb, 48)], probe)
            off = i - sb
            ob = jnp.minimum(off, 32)
            return extract(probe[pl.ds(ob, 16)], off - ob)

        start = find_boundary(w * cw)
        end = find_boundary((w + 1) * cw)
        astart = pl.multiple_of(start // 8 * 8, 8)
        nchunks = (end - astart + r - 1) // r

        def chunk_lb(k):
            cb = astart + k * r
            return cb, pl.multiple_of(jnp.minimum(cb, n - r), 8)

        def issue_chunk(k, par):
            _, lb = chunk_lb(k)
            pltpu.async_copy(h_hbm.at[pl.ds(lb, r)], rbufs[par], csems[par])
            pltpu.async_copy(
                ids_hbm.at[pl.ds(lb, r)], ibufs[par].at[pl.ds(0, r)],
                csems[par],
            )

        def wait_chunk(k, par):
            _, lb = chunk_lb(k)
            pltpu.make_async_copy(
                h_hbm.at[pl.ds(lb, r)], rbufs[par], csems[par]
            ).wait()
            pltpu.make_async_copy(
                ids_hbm.at[pl.ds(lb, r)], ibufs[par].at[pl.ds(0, r)],
                csems[par],
            ).wait()

        def flush_wait():
            pltpu.make_async_copy(stats_hbm.at[0], stage.at[1], fsem).wait()

        def flush(seg, cnt, sums, maxs):
            flush_wait()
            cv = jnp.zeros((16,), jnp.float32) + cnt.astype(jnp.float32)
            for i in range(nv):
                stage[0, pl.ds(i * 16, 16)] = sums[i]
                stage[0, pl.ds(d + i * 16, 16)] = maxs[i]
                stage[0, pl.ds(2 * d + i * 16, 16)] = cv
            pltpu.async_copy(stage.at[0], stats_hbm.at[seg], fsem)

        def accum(rowb, j0, j1, sums, maxs):
            def rbody(j, vs):
                ss, mm = vs
                ss2 = []
                mm2 = []
                for i in range(nv):
                    v = rowb[j, pl.ds(i * 16, 16)]
                    ss2.append(ss[i] + v)
                    mm2.append(jnp.maximum(mm[i], v))
                return (tuple(ss2), tuple(mm2))

            return lax.fori_loop(j0, j1, rbody, (sums, maxs))

        zero_vs = tuple(jnp.zeros((16,), jnp.float32) for _ in range(nv))
        neg_vs = tuple(
            jnp.full((16,), -3.4e38, jnp.float32) for _ in range(nv)
        )

        def process_chunk(k, par, carry):
            cur, cnt, sums, maxs = carry
            wait_chunk(k, par)

            @pl.when(k + 1 < nchunks)
            def _():
                issue_chunk(k + 1, par ^ 1)

            cb, lb = chunk_lb(k)
            rowb = rbufs[par]
            idbuf = ibufs[par]
            jlo = jnp.maximum(start, cb) - lb
            jhi = jnp.minimum(end, cb + r) - lb

            jb = jnp.minimum(jlo, r - 16)
            idlo = extract(idbuf[pl.ds(jb, 16)], jlo - jb)

            def cross_flush(c):
                cur2, cnt2, ss, mm = c
                flush(cur2, cnt2, ss, mm)
                return (idlo, jnp.int32(0), zero_vs, neg_vs)

            cur, cnt, sums, maxs = lax.cond(
                idlo != cur, cross_flush, lambda c: c,
                (cur, cnt, sums, maxs),
            )

            j = jlo
            for gg in range(r // 16):
                p0 = gg * 16
                v = idbuf[pl.ds(p0 + 1, 16)]
                vp = idbuf[pl.ds(p0, 16)]
                idxv = iota16 + (p0 + 1)
                mask = (v != vp) & (idxv > jlo) & (idxv < jhi)
                plsc.store_compressed(cmpb.at[pl.ds(0, 16)], idxv, mask=mask)
                pc = jnp.max(
                    plsc.all_reduce_population_count(mask)
                ).astype(jnp.int32)

                def seg_body(bi, c):
                    cur2, cnt2, j2, ss, mm = c
                    b = extract(cmpb[pl.ds(jnp.minimum(bi, 15), 16)], 0)
                    ss, mm = accum(rowb, j2, b, ss, mm)
                    cnt2 = cnt2 + (b - j2)
                    flush(cur2, cnt2, ss, mm)
                    sid = extract(idbuf[pl.ds(b, 16)], 0)
                    return (sid, jnp.int32(0), b, zero_vs, neg_vs)

                cur, cnt, j, sums, maxs = lax.fori_loop(
                    0, pc, seg_body, (cur, cnt, j, sums, maxs)
                )

                g1 = jnp.minimum(jhi, p0 + 16)
                sums, maxs = accum(rowb, j, g1, sums, maxs)
                cnt = cnt + jnp.maximum(g1 - j, 0)
                j = jnp.maximum(j, g1)

            return (cur, cnt, sums, maxs)

        @pl.when(start < end)
        def _():
            cur0 = load_id(start)
            issue_chunk(0, 0)
            pltpu.async_copy(stats_hbm.at[0], stage.at[1], fsem)

            def pair(kk, carry):
                for par in (0, 1):
                    k = kk * 2 + par
                    carry = lax.cond(
                        k < nchunks,
                        functools.partial(process_chunk, k, par),
                        lambda c: c,
                        carry,
                    )
                return carry

            init = (cur0, jnp.int32(0), zero_vs, neg_vs)
            cur, cnt, sums, maxs = lax.fori_loop(
                0, (nchunks + 1) // 2, pair, init
            )
            flush(cur, cnt, sums, maxs)
            flush_wait()

    return body(h, ids)


def _gather(table, ids):
    n = ids.shape[0]
    s, d = table.shape
    nv = d // 16
    info = plsc.get_sparse_core_info()
    nc, ns = info.num_cores, info.num_subcores
    nw = nc * ns
    cw = n // nw
    t = 128
    wsz = 64
    nring = 4
    nch = (cw + t - 1) // t
    mesh = plsc.VectorSubcoreMesh(core_axis_name="c", subcore_axis_name="s")

    @functools.partial(
        pl.kernel,
        out_type=jax.ShapeDtypeStruct((n, d), jnp.float32),
        mesh=mesh,
        compiler_params=pltpu.CompilerParams(needs_layout_passes=False),
        scratch_types=[pltpu.VMEM((cw + 16,), jnp.int32)]
        + [pltpu.VMEM((wsz, d), jnp.float32) for _ in range(nring)]
        + [pltpu.VMEM((t, d), jnp.float32) for _ in range(nring)]
        + [pltpu.VMEM((32,), jnp.int32)]
        + [pltpu.SemaphoreType.DMA for _ in range(2 * nring + 1)],
    )
    def body(tab_hbm, ids_hbm, out_hbm, ixall, *bufs):
        winb = bufs[:nring]
        stgb = bufs[nring:2 * nring]
        cmpb = bufs[2 * nring]
        wsem = bufs[2 * nring + 1:3 * nring + 1]
        ssem = bufs[3 * nring + 1:4 * nring + 1]
        fbsem = bufs[4 * nring + 1]
        w = lax.axis_index("s") * nc + lax.axis_index("c")
        base0 = w * cw
        iota16 = lax.iota(jnp.int32, 16)

        def extract(v, off):
            return jnp.max(jnp.where(iota16 == off, v, jnp.int32(-1)))

        def lbase(k):
            return pl.multiple_of(jnp.minimum(k * t, cw - t), 8)

        def meta(k):
            j0 = lbase(k)
            idf = extract(ixall[pl.ds(j0, 16)], 0)
            idl = extract(ixall[pl.ds(j0 + t - 16, 16)], 15)
            wb = pl.multiple_of(jnp.minimum(idf, s - wsz) // 8 * 8, 8)
            ok = (idl - wb) < wsz
            return j0, wb, ok

        def issue_window(k, p):
            _, wb, ok = meta(k)

            @pl.when(ok)
            def _():
                pltpu.async_copy(
                    tab_hbm.at[pl.ds(wb, wsz)], winb[p], wsem[p]
                )

        def run_copy(stage, win, a, b, src):
            regs = [win[src, pl.ds(i * 16, 16)] for i in range(nv)]

            def rbody(jr, _):
                for i in range(nv):
                    stage[jr, pl.ds(i * 16, 16)] = regs[i]
                return 0

            lax.fori_loop(a, b, rbody, 0)

        def step(k, p):
            @pl.when(k >= nring)
            def _():
                pltpu.make_async_copy(
                    stgb[p], out_hbm.at[pl.ds(base0 + lbase(k - nring), t)],
                    ssem[p],
                ).wait()

            j0, wb, ok = meta(k)

            def expand(_):
                pltpu.make_async_copy(
                    tab_hbm.at[pl.ds(wb, wsz)], winb[p], wsem[p]
                ).wait()
                cursrc = extract(ixall[pl.ds(j0, 16)], 0) - wb
                j = jnp.int32(0)
                carry = (j, cursrc)
                for q8 in range(t // 16):
                    p1 = j0 + q8 * 16 + 1
                    v = ixall[pl.ds(p1, 16)]
                    vp = ixall[pl.ds(p1 - 1, 16)]
                    idxv = iota16 + (q8 * 16 + 1)
                    mask = (v != vp) & (idxv < t)
                    plsc.store_compressed(
                        cmpb.at[pl.ds(0, 16)], idxv, mask=mask
                    )
                    pc = jnp.max(
                        plsc.all_reduce_population_count(mask)
                    ).astype(jnp.int32)

                    def rloop(bi, c):
                        j2, src2 = c
                        b = extract(
                            cmpb[pl.ds(jnp.minimum(bi, 15), 16)], 0
                        )
                        run_copy(stgb[p], winb[p], j2, b, src2)
                        nsrc = extract(ixall[pl.ds(j0 + b, 16)], 0) - wb
                        return (b, nsrc)

                    carry = lax.fori_loop(0, pc, rloop, carry)
                j, cursrc = carry
                run_copy(stgb[p], winb[p], j, t, cursrc)
                return 0

            def fallback(_):
                pltpu.async_copy(
                    tab_hbm.at[ixall.at[pl.ds(j0, t)]], stgb[p], fbsem
                ).wait()
                return 0

            lax.cond(ok, expand, fallback, 0)
            pltpu.async_copy(
                stgb[p], out_hbm.at[pl.ds(base0 + j0, t)], ssem[p]
            )

            @pl.when(k + 3 < nch)
            def _():
                issue_window(k + 3, (p + 3) % nring)

        pltpu.sync_copy(ids_hbm.at[pl.ds(base0, cw)], ixall.at[pl.ds(0, cw)])

        for kp in range(min(3, nch)):
            issue_window(kp, kp)

        def ring(kk, _):
            for par in range(nring):
                k = kk * nring + par

                @pl.when(k < nch)
                def _():
                    step(k, par)

            return 0

        lax.fori_loop(0, (nch + nring - 1) // nring, ring, 0)
        for tail in range(max(nch - nring, 0), nch):
            pltpu.make_async_copy(
                stgb[tail % nring],
                out_hbm.at[pl.ds(base0 + lbase(tail), t)],
                ssem[tail % nring],
            ).wait()

    return body(table, ids)


def kernel(x, batch_index, Wpre, bpre, Wproj, bproj):
    ids = batch_index.astype(jnp.int32)
    h = _prepool(x, Wpre.T, bpre)
    stats = _segstats(h, ids)
    pooled = _proj(stats, Wproj.T, bproj)
    return _gather(pooled, ids)


# speedup vs baseline: 1.2395x; 1.0139x over previous
import functools

import jax
import jax.numpy as jnp
from jax import lax
from jax.experimental import pallas as pl
from jax.experimental.pallas import tpu as pltpu
from jax.experimental.pallas import tpu_sc as plsc

_S = 10000
_D = 128


def _prepool_body(x_ref, wt_ref, b_ref, h_ref):
    h_ref[...] = (
        jnp.dot(x_ref[...], wt_ref[...], preferred_element_type=jnp.float32)
        + b_ref[...]
    )


def _prepool(x, wt, b):
    n, d = x.shape
    blk = 12800
    return pl.pallas_call(
        _prepool_body,
        grid=(n // blk,),
        in_specs=[
            pl.BlockSpec((blk, d), lambda i: (i, 0)),
            pl.BlockSpec((d, d), lambda i: (0, 0)),
            pl.BlockSpec((1, d), lambda i: (0, 0)),
        ],
        out_specs=pl.BlockSpec((blk, d), lambda i: (i, 0)),
        out_shape=jax.ShapeDtypeStruct((n, d), jnp.float32),
    )(x, wt, b.reshape(1, d))


def _proj_body(st_ref, wt_ref, b_ref, o_ref):
    st = st_ref[...]
    mean = st[:, 0:128] * (1.0 / jnp.maximum(st[:, 256:384], 1.0))
    xpool = jnp.concatenate([mean, st[:, 128:256]], axis=1)
    o_ref[...] = (
        jnp.dot(xpool, wt_ref[...], preferred_element_type=jnp.float32)
        + b_ref[...]
    )


def _proj(stats, wt, b):
    s = stats.shape[0]
    blk = 5000
    return pl.pallas_call(
        _proj_body,
        grid=(s // blk,),
        in_specs=[
            pl.BlockSpec((blk, 3 * _D), lambda i: (i, 0)),
            pl.BlockSpec((2 * _D, _D), lambda i: (0, 0)),
            pl.BlockSpec((1, _D), lambda i: (0, 0)),
        ],
        out_specs=pl.BlockSpec((blk, _D), lambda i: (i, 0)),
        out_shape=jax.ShapeDtypeStruct((s, _D), jnp.float32),
    )(stats, wt, b.reshape(1, _D))


def _segstats(h, ids):
    n, d = h.shape
    nv = d // 16
    st = 3 * d
    info = plsc.get_sparse_core_info()
    nc, ns = info.num_cores, info.num_subcores
    nw = nc * ns
    cw = n // nw
    r = 256
    mesh = plsc.VectorSubcoreMesh(core_axis_name="c", subcore_axis_name="s")

    @functools.partial(
        pl.kernel,
        out_type=jax.ShapeDtypeStruct((_S, st), jnp.float32),
        mesh=mesh,
        compiler_params=pltpu.CompilerParams(needs_layout_passes=False),
        scratch_types=[
            pltpu.VMEM((r, d), jnp.float32),
            pltpu.VMEM((r, d), jnp.float32),
            pltpu.VMEM((r + 16,), jnp.int32),
            pltpu.VMEM((r + 16,), jnp.int32),
            pltpu.VMEM((32,), jnp.int32),
            pltpu.VMEM((2, st), jnp.float32),
            pltpu.VMEM((48,), jnp.int32),
            pltpu.SemaphoreType.DMA,
            pltpu.SemaphoreType.DMA,
            pltpu.SemaphoreType.DMA,
        ],
    )
    def body(h_hbm, ids_hbm, stats_hbm, rb0, rb1, ib0, ib1, cmpb, stage,
             probe, csem0, csem1, fsem):
        w = lax.axis_index("s") * nc + lax.axis_index("c")
        iota16 = lax.iota(jnp.int32, 16)
        rbufs = (rb0, rb1)
        ibufs = (ib0, ib1)
        csems = (csem0, csem1)

        def extract(v, off):
            return jnp.max(jnp.where(iota16 == off, v, jnp.int32(-1)))

        def find_boundary(t):
            def b_cond(stt):
                base, res = stt
                return (res < 0) & (base < n)

            def b_body(stt):
                base, _ = stt
                a = pl.multiple_of(
                    jnp.maximum(jnp.minimum((base - 1) // 8 * 8, n - 48), 0), 8
                )
                pltpu.sync_copy(ids_hbm.at[pl.ds(a, 48)], probe)
                j = jnp.minimum(base - a, 32)
                v = probe[pl.ds(j, 16)]
                vp = probe[pl.ds(j - 1, 16)]
                idxv = a + j + iota16
                cand = jnp.where(
                    (v != vp) & (idxv >= base) & (idxv < n), idxv, n
                )
                m = jnp.min(cand)
                res2 = jnp.where(
                    m < n, m, jnp.where(base + 16 >= n, n, -1)
                ).astype(jnp.int32)
                return (jnp.int32(base + 16), res2)

            _, res = lax.while_loop(
                b_cond,
                b_body,
                (jnp.maximum(jnp.int32(t), 1), jnp.int32(-1)),
            )
            return jnp.where(
                t <= 0, 0, jnp.where(t >= n, n, res)
            ).astype(jnp.int32)

        def load_id(i):
            sb = pl.multiple_of(jnp.minimum(i // 8 * 8, n - 48), 8)
            pltpu.sync_copy(ids_hbm.at[pl.ds(sb, 48)], probe)
            off = i - sb
            ob = jnp.minimum(off, 32)
            return extract(probe[pl.ds(ob, 16)], off - ob)

        start = find_boundary(w * cw)
        end = find_boundary((w + 1) * cw)
        astart = pl.multiple_of(start // 8 * 8, 8)
        nchunks = (end - astart + r - 1) // r

        def chunk_lb(k):
            cb = astart + k * r
            return cb, pl.multiple_of(jnp.minimum(cb, n - r), 8)

        def issue_chunk(k, par):
            _, lb = chunk_lb(k)
            pltpu.async_copy(h_hbm.at[pl.ds(lb, r)], rbufs[par], csems[par])
            pltpu.async_copy(
                ids_hbm.at[pl.ds(lb, r)], ibufs[par].at[pl.ds(0, r)],
                csems[par],
            )

        def wait_chunk(k, par):
            _, lb = chunk_lb(k)
            pltpu.make_async_copy(
                h_hbm.at[pl.ds(lb, r)], rbufs[par], csems[par]
            ).wait()
            pltpu.make_async_copy(
                ids_hbm.at[pl.ds(lb, r)], ibufs[par].at[pl.ds(0, r)],
                csems[par],
            ).wait()

        def flush_wait():
            pltpu.make_async_copy(stats_hbm.at[0], stage.at[1], fsem).wait()

        def flush(seg, cnt, sums, maxs):
            flush_wait()
            cv = jnp.zeros((16,), jnp.float32) + cnt.astype(jnp.float32)
            for i in range(nv):
                stage[0, pl.ds(i * 16, 16)] = sums[i]
                stage[0, pl.ds(d + i * 16, 16)] = maxs[i]
                stage[0, pl.ds(2 * d + i * 16, 16)] = cv
            pltpu.async_copy(stage.at[0], stats_hbm.at[seg], fsem)

        def accum(rowb, j0, j1, sums, maxs):
            def rbody(j, vs):
                ss, mm = vs
                ss2 = []
                mm2 = []
                for i in range(nv):
                    v = rowb[j, pl.ds(i * 16, 16)]
                    ss2.append(ss[i] + v)
                    mm2.append(jnp.maximum(mm[i], v))
                return (tuple(ss2), tuple(mm2))

            return lax.fori_loop(j0, j1, rbody, (sums, maxs))

        zero_vs = tuple(jnp.zeros((16,), jnp.float32) for _ in range(nv))
        neg_vs = tuple(
            jnp.full((16,), -3.4e38, jnp.float32) for _ in range(nv)
        )

        def process_chunk(k, par, carry):
            cur, cnt, sums, maxs = carry
            wait_chunk(k, par)

            @pl.when(k + 1 < nchunks)
            def _():
                issue_chunk(k + 1, par ^ 1)

            cb, lb = chunk_lb(k)
            rowb = rbufs[par]
            idbuf = ibufs[par]
            jlo = jnp.maximum(start, cb) - lb
            jhi = jnp.minimum(end, cb + r) - lb

            jb = jnp.minimum(jlo, r - 16)
            idlo = extract(idbuf[pl.ds(jb, 16)], jlo - jb)

            def cross_flush(c):
                cur2, cnt2, ss, mm = c
                flush(cur2, cnt2, ss, mm)
                return (idlo, jnp.int32(0), zero_vs, neg_vs)

            cur, cnt, sums, maxs = lax.cond(
                idlo != cur, cross_flush, lambda c: c,
                (cur, cnt, sums, maxs),
            )

            j = jlo
            for gg in range(r // 16):
                p0 = gg * 16
                v = idbuf[pl.ds(p0 + 1, 16)]
                vp = idbuf[pl.ds(p0, 16)]
                idxv = iota16 + (p0 + 1)
                mask = (v != vp) & (idxv > jlo) & (idxv < jhi)
                plsc.store_compressed(cmpb.at[pl.ds(0, 16)], idxv, mask=mask)
                pc = jnp.max(
                    plsc.all_reduce_population_count(mask)
                ).astype(jnp.int32)

                def seg_body(bi, c):
                    cur2, cnt2, j2, ss, mm = c
                    b = extract(cmpb[pl.ds(jnp.minimum(bi, 15), 16)], 0)
                    ss, mm = accum(rowb, j2, b, ss, mm)
                    cnt2 = cnt2 + (b - j2)
                    flush(cur2, cnt2, ss, mm)
                    sid = extract(idbuf[pl.ds(b, 16)], 0)
                    return (sid, jnp.int32(0), b, zero_vs, neg_vs)

                cur, cnt, j, sums, maxs = lax.fori_loop(
                    0, pc, seg_body, (cur, cnt, j, sums, maxs)
                )

                g1 = jnp.minimum(jhi, p0 + 16)
                sums, maxs = accum(rowb, j, g1, sums, maxs)
                cnt = cnt + jnp.maximum(g1 - j, 0)
                j = jnp.maximum(j, g1)

            return (cur, cnt, sums, maxs)

        @pl.when(start < end)
        def _():
            cur0 = load_id(start)
            issue_chunk(0, 0)
            pltpu.async_copy(stats_hbm.at[0], stage.at[1], fsem)

            def pair(kk, carry):
                for par in (0, 1):
                    k = kk * 2 + par
                    carry = lax.cond(
                        k < nchunks,
                        functools.partial(process_chunk, k, par),
                        lambda c: c,
                        carry,
                    )
                return carry

            init = (cur0, jnp.int32(0), zero_vs, neg_vs)
            cur, cnt, sums, maxs = lax.fori_loop(
                0, (nchunks + 1) // 2, pair, init
            )
            flush(cur, cnt, sums, maxs)
            flush_wait()

    return body(h, ids)


def _gather(table, ids):
    n = ids.shape[0]
    s, d = table.shape
    nv = d // 16
    info = plsc.get_sparse_core_info()
    nc, ns = info.num_cores, info.num_subcores
    nw = nc * ns
    cw = n // nw
    t = 128
    wsz = 64
    nring = 4
    nch = (cw + t - 1) // t
    mesh = plsc.VectorSubcoreMesh(core_axis_name="c", subcore_axis_name="s")

    @functools.partial(
        pl.kernel,
        out_type=jax.ShapeDtypeStruct((n, d), jnp.float32),
        mesh=mesh,
        compiler_params=pltpu.CompilerParams(needs_layout_passes=False),
        scratch_types=[pltpu.VMEM((cw + 16,), jnp.int32)]
        + [pltpu.VMEM((wsz, d), jnp.float32) for _ in range(nring)]
        + [pltpu.VMEM((t, d), jnp.float32) for _ in range(nring)]
        + [pltpu.VMEM((32,), jnp.int32)]
        + [pltpu.SemaphoreType.DMA for _ in range(2 * nring + 1)],
    )
    def body(tab_hbm, ids_hbm, out_hbm, ixall, *bufs):
        winb = bufs[:nring]
        stgb = bufs[nring:2 * nring]
        cmpb = bufs[2 * nring]
        wsem = bufs[2 * nring + 1:3 * nring + 1]
        ssem = bufs[3 * nring + 1:4 * nring + 1]
        fbsem = bufs[4 * nring + 1]
        w = lax.axis_index("s") * nc + lax.axis_index("c")
        base0 = w * cw
        iota16 = lax.iota(jnp.int32, 16)

        def extract(v, off):
            return jnp.max(jnp.where(iota16 == off, v, jnp.int32(-1)))

        def lbase(k):
            return pl.multiple_of(jnp.minimum(k * t, cw - t), 8)

        def meta(k):
            j0 = lbase(k)
            idf = extract(ixall[pl.ds(j0, 16)], 0)
            idl = extract(ixall[pl.ds(j0 + t - 16, 16)], 15)
            wb = pl.multiple_of(jnp.minimum(idf, s - wsz) // 8 * 8, 8)
            ok = (idl - wb) < wsz
            return j0, wb, ok

        def issue_window(k, p):
            _, wb, ok = meta(k)

            @pl.when(ok)
            def _():
                pltpu.async_copy(
                    tab_hbm.at[pl.ds(wb, wsz)], winb[p], wsem[p]
                )

        def run_copy(stage, win, a, b, src):
            regs = [win[src, pl.ds(i * 16, 16)] for i in range(nv)]

            def rbody(jr, _):
                for i in range(nv):
                    stage[jr, pl.ds(i * 16, 16)] = regs[i]
                return 0

            lax.fori_loop(a, b, rbody, 0)

        def step(k, p):
            @pl.when(k >= nring)
            def _():
                pltpu.make_async_copy(
                    stgb[p], out_hbm.at[pl.ds(base0 + lbase(k - nring), t)],
                    ssem[p],
                ).wait()

            j0, wb, ok = meta(k)

            def expand(_):
                pltpu.make_async_copy(
                    tab_hbm.at[pl.ds(wb, wsz)], winb[p], wsem[p]
                ).wait()
                cursrc = extract(ixall[pl.ds(j0, 16)], 0) - wb
                j = jnp.int32(0)
                carry = (j, cursrc)
                for q8 in range(t // 16):
                    p1 = j0 + q8 * 16 + 1
                    v = ixall[pl.ds(p1, 16)]
                    vp = ixall[pl.ds(p1 - 1, 16)]
                    idxv = iota16 + (q8 * 16 + 1)
                    mask = (v != vp) & (idxv < t)
                    plsc.store_compressed(
                        cmpb.at[pl.ds(0, 16)], idxv, mask=mask
                    )
                    pc = jnp.max(
                        plsc.all_reduce_population_count(mask)
                    ).astype(jnp.int32)

                    def rloop(bi, c):
                        j2, src2 = c
                        b = extract(
                            cmpb[pl.ds(jnp.minimum(bi, 15), 16)], 0
                        )
                        run_copy(stgb[p], winb[p], j2, b, src2)
                        nsrc = extract(ixall[pl.ds(j0 + b, 16)], 0) - wb
                        return (b, nsrc)

                    carry = lax.fori_loop(0, pc, rloop, carry)
                j, cursrc = carry
                run_copy(stgb[p], winb[p], j, t, cursrc)
                return 0

            def fallback(_):
                pltpu.async_copy(
                    tab_hbm.at[ixall.at[pl.ds(j0, t)]], stgb[p], fbsem
                ).wait()
                return 0

            lax.cond(ok, expand, fallback, 0)
            pltpu.async_copy(
                stgb[p], out_hbm.at[pl.ds(base0 + j0, t)], ssem[p]
            )

            @pl.when(k + 3 < nch)
            def _():
                issue_window(k + 3, (p + 3) % nring)

        pltpu.sync_copy(ids_hbm.at[pl.ds(base0, cw)], ixall.at[pl.ds(0, cw)])

        for kp in range(min(3, nch)):
            issue_window(kp, kp)

        def ring(kk, _):
            for par in range(nring):
                k = kk * nring + par

                @pl.when(k < nch)
                def _():
                    step(k, par)

            return 0

        lax.fori_loop(0, (nch + nring - 1) // nring, ring, 0)
        for tail in range(max(nch - nring, 0), nch):
            pltpu.make_async_copy(
                stgb[tail % nring],
                out_hbm.at[pl.ds(base0 + lbase(tail), t)],
                ssem[tail % nring],
            ).wait()

    return body(table, ids)


def kernel(x, batch_index, Wpre, bpre, Wproj, bproj):
    ids = batch_index.astype(jnp.int32)
    h = _prepool(x, Wpre.T, bpre)
    stats = _segstats(h, ids)
    pooled = _proj(stats, Wproj.T, bproj)
    return _gather(pooled, ids)
